# baseline (speedup 1.0000x reference)
import jax
import jax.numpy as jnp
from jax.experimental import pallas as pl
from jax.experimental.pallas import tpu as pltpu


def _round_up(n, m):
    return ((n + m - 1) // m) * m


_CH = 1024
_NBUF = 3


def _make_stream_kernel(half, nch):
    def body(x_hbm, w_ref, b_ref, o_hbm, x_buf, o_buf, in_sem, out_sem):
        row0 = pl.program_id(0) * half

        def dma_in(slot, step):
            return pltpu.make_async_copy(
                x_hbm.at[pl.ds(row0 + step * _CH, _CH), :],
                x_buf.at[slot], in_sem.at[slot])

        def dma_out(slot, step):
            return pltpu.make_async_copy(
                o_buf.at[slot],
                o_hbm.at[pl.ds(row0 + step * _CH, _CH), :],
                out_sem.at[slot])

        for s in range(min(_NBUF, nch)):
            dma_in(s, s).start()

        wb = w_ref[...].astype(jnp.bfloat16)
        brow = b_ref[...]

        for step in range(nch):
            slot = step % _NBUF
            dma_in(slot, step).wait()
            xb = x_buf[slot].astype(jnp.bfloat16)
            acc = jax.lax.dot_general(
                xb, wb, (((1,), (1,)), ((), ())),
                preferred_element_type=jnp.float32)
            if step >= _NBUF:
                dma_out(slot, step - _NBUF).wait()
            o_buf[slot] = acc + brow
            dma_out(slot, step).start()
            if step + _NBUF < nch:
                dma_in(slot, step + _NBUF).start()

        for step in range(max(0, nch - _NBUF), nch):
            dma_out(step % _NBUF, step).wait()

    return body


def kernel(x, weight, bias):
    B, D = x.shape
    C, D2 = weight.shape
    assert D == D2 and bias.shape == (C,)

    CPAD = _round_up(C, 128)
    B_pad = _round_up(B, 2 * _CH)
    half = B_pad // 2
    nch = half // _CH

    x = x.astype(jnp.float32)
    x_p = x if B_pad == B else jnp.pad(x, ((0, B_pad - B), (0, 0)))
    w_p = weight.astype(jnp.float32)
    if CPAD != C:
        w_p = jnp.pad(w_p, ((0, CPAD - C), (0, 0)))
    b_row = jnp.pad(bias.astype(jnp.float32), (0, CPAD - C)).reshape(1, CPAD)

    cost = pl.CostEstimate(
        flops=2 * B * D * C,
        transcendentals=0,
        bytes_accessed=int(B_pad * D * 4 + D * CPAD * 4
                           + CPAD * 4 + B_pad * CPAD * 4),
    )

    out_padded = pl.pallas_call(
        _make_stream_kernel(half, nch),
        out_shape=jax.ShapeDtypeStruct((B_pad, CPAD), jnp.float32),
        grid_spec=pltpu.PrefetchScalarGridSpec(
            num_scalar_prefetch=0,
            grid=(2,),
            in_specs=[
                pl.BlockSpec(memory_space=pl.ANY),
                pl.BlockSpec((CPAD, D), lambda i: (0, 0)),
                pl.BlockSpec((1, CPAD), lambda i: (0, 0)),
            ],
            out_specs=pl.BlockSpec(memory_space=pl.ANY),
            scratch_shapes=[
                pltpu.VMEM((_NBUF, _CH, D), jnp.float32),
                pltpu.VMEM((_NBUF, _CH, CPAD), jnp.float32),
                pltpu.SemaphoreType.DMA((_NBUF,)),
                pltpu.SemaphoreType.DMA((_NBUF,)),
            ],
        ),
        compiler_params=pltpu.CompilerParams(
            dimension_semantics=("parallel",),
            vmem_limit_bytes=56 * 1024 * 1024),
        cost_estimate=cost,
    )(x_p, w_p, b_row)

    return out_padded[:B, :C]
